# Initial kernel scaffold; baseline (speedup 1.0000x reference)
import jax
import jax.numpy as jnp
from jax.experimental import pallas as pl


def kernel(x, edge_index, W_enc, W_dec, last_h_storage):
    raise NotImplementedError("write your pallas kernel here")


# speedup vs baseline: 3.4860x; 3.4860x over previous
import functools

import jax
import jax.numpy as jnp
from jax import lax
from jax.experimental import pallas as pl
from jax.experimental.pallas import tpu as pltpu
from jax.experimental.pallas import tpu_sc as plsc

N_NODES = 100000
DIM = 128
N_EDGES = 200000
N_CORES = 2
N_SUB = 16
NW = N_CORES * N_SUB
CHUNK = 128
CPW = 49
E_PAD = NW * CPW * CHUNK
EPW = 2 * E_PAD // NW
MCPW = EPW // CHUNK
ROW_BLOCK = 2000

_vmesh = plsc.VectorSubcoreMesh(core_axis_name="c", subcore_axis_name="s")
_sc_params = pltpu.CompilerParams(needs_layout_passes=False)


def _z():
    return jnp.int32(0)


@functools.partial(
    pl.kernel,
    out_type=jax.ShapeDtypeStruct((NW * N_NODES,), jnp.float32),
    mesh=_vmesh,
    scratch_types=[
        pltpu.VMEM((N_NODES,), jnp.float32),
        pltpu.VMEM((CHUNK,), jnp.int32),
    ],
    compiler_params=_sc_params,
)
def _mask_kernel(idx_hbm, zeros_hbm, mask_hbm, mask_v, idx_v):
    i32 = jnp.int32
    c = lax.axis_index("c")
    s = lax.axis_index("s")
    w = s * i32(N_CORES) + c
    pltpu.sync_copy(zeros_hbm, mask_v)
    ones16 = jnp.full((16,), 1.0, jnp.float32)

    @pl.loop(0, MCPW, step=jnp.int32(1))
    def _(j):
        base = (w * i32(MCPW) + j) * i32(CHUNK)
        pltpu.sync_copy(idx_hbm.at[pl.ds(base, CHUNK)], idx_v)
        for e in range(CHUNK // 16):
            idx16 = idx_v[pl.ds(e * 16, 16)]
            plsc.store_scatter(mask_v, [idx16], ones16)

    pltpu.sync_copy(mask_v, mask_hbm.at[pl.ds(w * i32(N_NODES), N_NODES)])


@functools.partial(
    pl.kernel,
    out_type=jax.ShapeDtypeStruct((E_PAD * 16,), jnp.float32),
    mesh=_vmesh,
    scratch_types=[
        pltpu.VMEM((CHUNK,), jnp.int32),
        pltpu.VMEM((CHUNK,), jnp.int32),
        pltpu.VMEM((CHUNK, DIM), jnp.float32),
        pltpu.VMEM((CHUNK, DIM), jnp.float32),
        pltpu.VMEM((CHUNK * 16,), jnp.float32),
        pltpu.SemaphoreType.DMA,
        pltpu.SemaphoreType.DMA,
    ],
    compiler_params=_sc_params,
)
def _score_kernel(g_hbm, h_hbm, src_hbm, dst_hbm, out_hbm,
                  sidx, didx, gs, hd, sc_v, sem1, sem2):
    i32 = jnp.int32
    c = lax.axis_index("c")
    s = lax.axis_index("s")
    w = s * i32(N_CORES) + c

    @pl.loop(0, CPW, step=jnp.int32(1))
    def _(j):
        base = (w * i32(CPW) + j) * i32(CHUNK)
        pltpu.sync_copy(src_hbm.at[pl.ds(base, CHUNK)], sidx)
        pltpu.sync_copy(dst_hbm.at[pl.ds(base, CHUNK)], didx)
        cp1 = pltpu.async_copy(g_hbm.at[sidx], gs, sem1)
        cp2 = pltpu.async_copy(h_hbm.at[didx], hd, sem2)
        cp1.wait()
        cp2.wait()

        @pl.loop(0, CHUNK, step=jnp.int32(1))
        def _(e):
            acc = gs[e, pl.ds(0, 16)] * hd[e, pl.ds(0, 16)]
            for k in range(1, DIM // 16):
                acc = acc + gs[e, pl.ds(16 * k, 16)] * hd[e, pl.ds(16 * k, 16)]
            sc_v[pl.ds(e * i32(16), 16)] = acc

        pltpu.sync_copy(sc_v, out_hbm.at[pl.ds(base * i32(16), CHUNK * 16)])


def _mm_body(x_ref, we_ref, wd_ref, h_out, g_out):
    h = jnp.maximum(
        jnp.dot(x_ref[...], we_ref[...], preferred_element_type=jnp.float32,
                precision=lax.Precision.HIGHEST), 0.0)
    h_out[...] = h
    g_out[...] = jnp.dot(h, wd_ref[...], preferred_element_type=jnp.float32,
                         precision=lax.Precision.HIGHEST)


_mm_call = pl.pallas_call(
    _mm_body,
    grid=(N_NODES // ROW_BLOCK,),
    in_specs=[
        pl.BlockSpec((ROW_BLOCK, DIM), lambda i: (i, _z())),
        pl.BlockSpec((DIM, DIM), lambda i: (_z(), _z())),
        pl.BlockSpec((DIM, DIM), lambda i: (_z(), _z())),
    ],
    out_specs=[
        pl.BlockSpec((ROW_BLOCK, DIM), lambda i: (i, _z())),
        pl.BlockSpec((ROW_BLOCK, DIM), lambda i: (i, _z())),
    ],
    out_shape=[
        jax.ShapeDtypeStruct((N_NODES, DIM), jnp.float32),
        jax.ShapeDtypeStruct((N_NODES, DIM), jnp.float32),
    ],
    compiler_params=pltpu.CompilerParams(
        dimension_semantics=("parallel",)),
)


MB = 2048
M_GRID = -(-N_NODES // MB)


def _merge_body(m_ref, out_ref):
    ones = jnp.full((NW, 16), 1.0, jnp.float32)
    out_ref[...] = jnp.dot(m_ref[...].T, ones,
                           preferred_element_type=jnp.float32)


_merge_call = pl.pallas_call(
    _merge_body,
    grid=(M_GRID,),
    in_specs=[pl.BlockSpec((NW, MB), lambda i: (_z(), i))],
    out_specs=pl.BlockSpec((MB, 16), lambda i: (i, _z())),
    out_shape=jax.ShapeDtypeStruct((N_NODES, 16), jnp.float32),
    compiler_params=pltpu.CompilerParams(
        dimension_semantics=("parallel",)),
)


def _sel_body(h_ref, m_ref, last_ref, ns_out):
    touched = m_ref[...][:, :1] > 0.0
    ns_out[...] = jnp.where(touched, h_ref[...], last_ref[...])


_sel_call = pl.pallas_call(
    _sel_body,
    grid=(N_NODES // ROW_BLOCK,),
    in_specs=[
        pl.BlockSpec((ROW_BLOCK, DIM), lambda i: (i, _z())),
        pl.BlockSpec((ROW_BLOCK, 16), lambda i: (i, _z())),
        pl.BlockSpec((ROW_BLOCK, DIM), lambda i: (i, _z())),
    ],
    out_specs=pl.BlockSpec((ROW_BLOCK, DIM), lambda i: (i, _z())),
    out_shape=jax.ShapeDtypeStruct((N_NODES, DIM), jnp.float32),
    compiler_params=pltpu.CompilerParams(
        dimension_semantics=("parallel",)),
)


LOSS_BLK = E_PAD // 32


def _loss_body(p_ref, out_ref):
    i = pl.program_id(0)
    part = p_ref[...]
    score = jnp.sum(part, axis=1, keepdims=True)
    z = -score
    sp = jnp.maximum(z, 0.0) + jnp.log1p(jnp.exp(-jnp.abs(z)))
    rows = lax.broadcasted_iota(jnp.int32, sp.shape, 0) + i * LOSS_BLK
    sp = jnp.where(rows < N_EDGES, sp, 0.0)
    val = jnp.sum(sp) * (1.0 / N_EDGES)

    @pl.when(i == 0)
    def _():
        out_ref[...] = jnp.zeros_like(out_ref)

    out_ref[...] += jnp.full((1, 1), 1.0, jnp.float32) * val


_loss_call = pl.pallas_call(
    _loss_body,
    grid=(E_PAD // LOSS_BLK,),
    in_specs=[pl.BlockSpec((LOSS_BLK, 16), lambda i: (i, _z()))],
    out_specs=pl.BlockSpec((1, 1), lambda i: (_z(), _z())),
    out_shape=jax.ShapeDtypeStruct((1, 1), jnp.float32),
)


def kernel(x, edge_index, W_enc, W_dec, last_h_storage):
    x = x.astype(jnp.float32)
    src = edge_index[0].astype(jnp.int32)
    dst = edge_index[1].astype(jnp.int32)
    pad = E_PAD - N_EDGES
    src_p = jnp.concatenate([src, jnp.broadcast_to(src[:1], (pad,))])
    dst_p = jnp.concatenate([dst, jnp.broadcast_to(dst[:1], (pad,))])
    idx_flat = jnp.concatenate([src_p, dst_p])
    zeros_c = jnp.zeros((N_NODES,), jnp.float32)

    mask_parts = _mask_kernel(idx_flat, zeros_c).reshape(NW, N_NODES)
    mask16 = _merge_call(mask_parts)
    h, g = _mm_call(x, W_enc.astype(jnp.float32), W_dec.astype(jnp.float32))
    new_storage = _sel_call(h, mask16, last_h_storage.astype(jnp.float32))
    parts = _score_kernel(g, h, src_p, dst_p)
    loss = _loss_call(parts.reshape(E_PAD, 16)).reshape(1)
    return loss, new_storage


# speedup vs baseline: 4.7663x; 1.3673x over previous
import functools

import jax
import jax.numpy as jnp
from jax import lax
from jax.experimental import pallas as pl
from jax.experimental.pallas import tpu as pltpu
from jax.experimental.pallas import tpu_sc as plsc

N_NODES = 100000
DIM = 128
N_EDGES = 200000
N_CORES = 2
N_SUB = 16
NW = N_CORES * N_SUB
CHUNK = 128
CPW = 49
E_PAD = NW * CPW * CHUNK
EPW = 2 * E_PAD // NW
MCPW = EPW // CHUNK
ROW_BLOCK = 2000

_vmesh = plsc.VectorSubcoreMesh(core_axis_name="c", subcore_axis_name="s")
_sc_params = pltpu.CompilerParams(needs_layout_passes=False)


def _z():
    return jnp.int32(0)


@functools.partial(
    pl.kernel,
    out_type=jax.ShapeDtypeStruct((NW * N_NODES,), jnp.float32),
    mesh=_vmesh,
    scratch_types=[
        pltpu.VMEM((N_NODES,), jnp.float32),
        pltpu.VMEM((CHUNK,), jnp.int32),
    ],
    compiler_params=_sc_params,
)
def _mask_kernel(idx_hbm, zeros_hbm, mask_hbm, mask_v, idx_v):
    i32 = jnp.int32
    c = lax.axis_index("c")
    s = lax.axis_index("s")
    w = s * i32(N_CORES) + c
    pltpu.sync_copy(zeros_hbm, mask_v)
    ones16 = jnp.full((16,), 1.0, jnp.float32)

    @pl.loop(0, MCPW, step=jnp.int32(1))
    def _(j):
        base = (w * i32(MCPW) + j) * i32(CHUNK)
        pltpu.sync_copy(idx_hbm.at[pl.ds(base, CHUNK)], idx_v)
        for e in range(CHUNK // 16):
            idx16 = idx_v[pl.ds(e * 16, 16)]
            plsc.store_scatter(mask_v, [idx16], ones16)

    pltpu.sync_copy(mask_v, mask_hbm.at[pl.ds(w * i32(N_NODES), N_NODES)])


@functools.partial(
    pl.kernel,
    out_type=jax.ShapeDtypeStruct((E_PAD * 16,), jnp.float32),
    mesh=_vmesh,
    scratch_types=[
        pltpu.VMEM((CHUNK,), jnp.int32),
        pltpu.VMEM((CHUNK,), jnp.int32),
        pltpu.VMEM((CHUNK, DIM), jnp.float32),
        pltpu.VMEM((CHUNK, DIM), jnp.float32),
        pltpu.VMEM((CHUNK * 16,), jnp.float32),
        pltpu.SemaphoreType.DMA,
        pltpu.SemaphoreType.DMA,
    ],
    compiler_params=_sc_params,
)
def _score_kernel(g_hbm, h_hbm, src_hbm, dst_hbm, out_hbm,
                  sidx, didx, gs, hd, sc_v, sem1, sem2):
    i32 = jnp.int32
    c = lax.axis_index("c")
    s = lax.axis_index("s")
    w = s * i32(N_CORES) + c

    @pl.loop(0, CPW, step=jnp.int32(1))
    def _(j):
        base = (w * i32(CPW) + j) * i32(CHUNK)
        pltpu.sync_copy(src_hbm.at[pl.ds(base, CHUNK)], sidx)
        pltpu.sync_copy(dst_hbm.at[pl.ds(base, CHUNK)], didx)
        cp1 = pltpu.async_copy(g_hbm.at[sidx], gs, sem1)
        cp2 = pltpu.async_copy(h_hbm.at[didx], hd, sem2)
        cp1.wait()
        cp2.wait()

        @pl.loop(0, CHUNK, step=jnp.int32(1))
        def _(e):
            acc = gs[e, pl.ds(0, 16)] * hd[e, pl.ds(0, 16)]
            for k in range(1, DIM // 16):
                acc = acc + gs[e, pl.ds(16 * k, 16)] * hd[e, pl.ds(16 * k, 16)]
            sc_v[pl.ds(e * i32(16), 16)] = acc

        pltpu.sync_copy(sc_v, out_hbm.at[pl.ds(base * i32(16), CHUNK * 16)])


def _mm_body(x_ref, we_ref, wd_ref, h_out, g_out):
    h = jnp.maximum(
        jnp.dot(x_ref[...], we_ref[...], preferred_element_type=jnp.float32),
        0.0)
    h_out[...] = h
    g_out[...] = jnp.dot(h, wd_ref[...], preferred_element_type=jnp.float32)


_mm_call = pl.pallas_call(
    _mm_body,
    grid=(N_NODES // ROW_BLOCK,),
    in_specs=[
        pl.BlockSpec((ROW_BLOCK, DIM), lambda i: (i, _z())),
        pl.BlockSpec((DIM, DIM), lambda i: (_z(), _z())),
        pl.BlockSpec((DIM, DIM), lambda i: (_z(), _z())),
    ],
    out_specs=[
        pl.BlockSpec((ROW_BLOCK, DIM), lambda i: (i, _z())),
        pl.BlockSpec((ROW_BLOCK, DIM), lambda i: (i, _z())),
    ],
    out_shape=[
        jax.ShapeDtypeStruct((N_NODES, DIM), jnp.float32),
        jax.ShapeDtypeStruct((N_NODES, DIM), jnp.float32),
    ],
    compiler_params=pltpu.CompilerParams(
        dimension_semantics=("parallel",)),
)


MB = 2048
M_GRID = -(-N_NODES // MB)


def _merge_body(m_ref, out_ref):
    ones = jnp.full((NW, 16), 1.0, jnp.float32)
    out_ref[...] = jnp.dot(m_ref[...].T, ones,
                           preferred_element_type=jnp.float32)


_merge_call = pl.pallas_call(
    _merge_body,
    grid=(M_GRID,),
    in_specs=[pl.BlockSpec((NW, MB), lambda i: (_z(), i))],
    out_specs=pl.BlockSpec((MB, 16), lambda i: (i, _z())),
    out_shape=jax.ShapeDtypeStruct((N_NODES, 16), jnp.float32),
    compiler_params=pltpu.CompilerParams(
        dimension_semantics=("parallel",)),
)


def _sel_body(h_ref, m_ref, last_ref, ns_out):
    touched = m_ref[...][:, :1] > 0.0
    ns_out[...] = jnp.where(touched, h_ref[...], last_ref[...])


_sel_call = pl.pallas_call(
    _sel_body,
    grid=(N_NODES // ROW_BLOCK,),
    in_specs=[
        pl.BlockSpec((ROW_BLOCK, DIM), lambda i: (i, _z())),
        pl.BlockSpec((ROW_BLOCK, 16), lambda i: (i, _z())),
        pl.BlockSpec((ROW_BLOCK, DIM), lambda i: (i, _z())),
    ],
    out_specs=pl.BlockSpec((ROW_BLOCK, DIM), lambda i: (i, _z())),
    out_shape=jax.ShapeDtypeStruct((N_NODES, DIM), jnp.float32),
    compiler_params=pltpu.CompilerParams(
        dimension_semantics=("parallel",)),
)


LOSS_ROWS = E_PAD * 16 // DIM
LOSS_BLK = LOSS_ROWS // 4
REAL_ROWS = N_EDGES // 8


def _loss_body(p_ref, out_ref):
    i = pl.program_id(0)
    part = p_ref[...]
    rows = lax.broadcasted_iota(jnp.int32, (LOSS_BLK, 1), 0) + i * LOSS_BLK
    valid = rows < REAL_ROWS
    total = jnp.zeros((), jnp.float32)
    for e in range(8):
        score = jnp.sum(part[:, 16 * e:16 * (e + 1)], axis=1, keepdims=True)
        z = -score
        sp = jnp.maximum(z, 0.0) + jnp.log1p(jnp.exp(-jnp.abs(z)))
        total = total + jnp.sum(jnp.where(valid, sp, 0.0))
    val = total * (1.0 / N_EDGES)

    @pl.when(i == 0)
    def _():
        out_ref[...] = jnp.zeros_like(out_ref)

    out_ref[...] += jnp.full((1, 1), 1.0, jnp.float32) * val


_loss_call = pl.pallas_call(
    _loss_body,
    grid=(LOSS_ROWS // LOSS_BLK,),
    in_specs=[pl.BlockSpec((LOSS_BLK, DIM), lambda i: (i, _z()))],
    out_specs=pl.BlockSpec((1, 1), lambda i: (_z(), _z())),
    out_shape=jax.ShapeDtypeStruct((1, 1), jnp.float32),
)


def kernel(x, edge_index, W_enc, W_dec, last_h_storage):
    x = x.astype(jnp.float32)
    src = edge_index[0].astype(jnp.int32)
    dst = edge_index[1].astype(jnp.int32)
    pad = E_PAD - N_EDGES
    src_p = jnp.concatenate([src, jnp.broadcast_to(src[:1], (pad,))])
    dst_p = jnp.concatenate([dst, jnp.broadcast_to(dst[:1], (pad,))])
    idx_flat = jnp.concatenate([src_p, dst_p])
    zeros_c = jnp.zeros((N_NODES,), jnp.float32)

    mask_parts = _mask_kernel(idx_flat, zeros_c).reshape(NW, N_NODES)
    mask16 = _merge_call(mask_parts)
    h, g = _mm_call(x, W_enc.astype(jnp.float32), W_dec.astype(jnp.float32))
    new_storage = _sel_call(h, mask16, last_h_storage.astype(jnp.float32))
    src_q = src_p + (mask_parts[0, :1] * 0.0).astype(jnp.int32)
    parts = _score_kernel(g, h, src_q, dst_p)
    loss = _loss_call(parts.reshape(LOSS_ROWS, DIM)).reshape(1)
    return loss, new_storage
